# Initial kernel scaffold; baseline (speedup 1.0000x reference)
import jax
import jax.numpy as jnp
from jax.experimental import pallas as pl


def kernel(conn_a, conn_b, nodes_a, nodes_b, W_e, b_e, W_n, b_n, W_l1, b_l1, W_l2, b_l2):
    raise NotImplementedError("write your pallas kernel here")


# speedup vs baseline: 3.9798x; 3.9798x over previous
import functools

import jax
import jax.numpy as jnp
from jax import lax
from jax.experimental import pallas as pl
from jax.experimental.pallas import tpu as pltpu

N = 2048
K = 256
DE = 16
DF = 128
H = 128
BI = 256
T = N // BI


def _qap_body(nodes_a_ref, nodes_b_ref, conn_a_ref, conn_b_ref,
              W_e_ref, b_e_ref, W_n_ref, b_n_ref, W_l1_ref, b_l1_ref,
              W_l2_ref, b_l2_ref, out_ref,
              oh_a_ref, oh_b_ref, acc_a_ref, acc_b_ref):
    step = pl.program_id(0)

    @pl.when(step == 0)
    def _init():
        j = lax.broadcasted_iota(jnp.int32, (N, K), 0)
        oh_a_ref[...] = (j == nodes_a_ref[...]).astype(jnp.bfloat16)
        oh_b_ref[...] = (j == nodes_b_ref[...]).astype(jnp.bfloat16)
        acc_a_ref[...] = jnp.zeros_like(acc_a_ref)
        acc_b_ref[...] = jnp.zeros_like(acc_b_ref)

    def accumulate(conn_ref, oh_ref, acc_ref):
        c = conn_ref[...].astype(jnp.bfloat16)
        g = jnp.dot(c, oh_ref[...],
                    preferred_element_type=jnp.float32)
        rows = []
        for d in range(DE):
            w = W_e_ref[0, d]
            b = b_e_ref[d]
            e = jnp.maximum(g * w + b, 0.0)
            rows.append(jnp.sum(e, axis=0, keepdims=True))
        acc_ref[...] += jnp.concatenate(rows, axis=0)

    accumulate(conn_a_ref, oh_a_ref, acc_a_ref)
    accumulate(conn_b_ref, oh_b_ref, acc_b_ref)

    @pl.when(step == T - 1)
    def _head():
        dn = (((0,), (0,)), ((), ()))
        base_a = jnp.maximum(
            lax.dot_general(acc_a_ref[...], W_n_ref[...], dn,
                            preferred_element_type=jnp.float32)
            + b_n_ref[...], 0.0)
        base_b = jnp.maximum(
            lax.dot_general(acc_b_ref[...], W_n_ref[...], dn,
                            preferred_element_type=jnp.float32)
            + b_n_ref[...], 0.0)
        w_a = W_l1_ref[0:DF, :]
        w_b = W_l1_ref[DF:2 * DF, :]
        pa = jnp.dot(base_a, w_a,
                     preferred_element_type=jnp.float32) + b_l1_ref[...]
        dnt = (((0,), (1,)), ((), ()))
        pbt = lax.dot_general(w_b, base_b, dnt,
                              preferred_element_type=jnp.float32)
        acc = jnp.full((K, K), b_l2_ref[0], dtype=jnp.float32)
        for f in range(H):
            hmat = jnp.maximum(pa[:, f:f + 1] + pbt[f:f + 1, :], 0.0)
            acc = acc + W_l2_ref[0, f] * hmat
        out_ref[...] = 1.0 / (1.0 + jnp.exp(-acc))


@functools.partial(jax.jit, static_argnames=())
def _run(conn_a, conn_b, na, nb, W_e, b_e, W_n, b_n2, W_l1, b_l1_2, w2row, b_l2):
    in_specs = [
            pl.BlockSpec((1, K), lambda i: (0, 0)),
            pl.BlockSpec((1, K), lambda i: (0, 0)),
            pl.BlockSpec((BI, N), lambda i: (i, 0)),
            pl.BlockSpec((BI, N), lambda i: (i, 0)),
            pl.BlockSpec(memory_space=pltpu.SMEM),
            pl.BlockSpec(memory_space=pltpu.SMEM),
            pl.BlockSpec((DE, DF), lambda i: (0, 0)),
            pl.BlockSpec((1, DF), lambda i: (0, 0)),
            pl.BlockSpec((2 * DF, H), lambda i: (0, 0)),
            pl.BlockSpec((1, H), lambda i: (0, 0)),
            pl.BlockSpec(memory_space=pltpu.SMEM),
            pl.BlockSpec(memory_space=pltpu.SMEM),
    ]
    return pl.pallas_call(
        _qap_body,
        grid=(T,),
        in_specs=in_specs,
        out_specs=pl.BlockSpec((K, K), lambda i: (0, 0)),
        out_shape=jax.ShapeDtypeStruct((K, K), jnp.float32),
        scratch_shapes=[
            pltpu.VMEM((N, K), jnp.bfloat16),
            pltpu.VMEM((N, K), jnp.bfloat16),
            pltpu.VMEM((DE, K), jnp.float32),
            pltpu.VMEM((DE, K), jnp.float32),
        ],
        compiler_params=pltpu.CompilerParams(
            dimension_semantics=("arbitrary",),
        ),
    )(na, nb, conn_a, conn_b, W_e, b_e, W_n, b_n2, W_l1, b_l1_2, w2row, b_l2)


def kernel(conn_a, conn_b, nodes_a, nodes_b, W_e, b_e, W_n, b_n,
           W_l1, b_l1, W_l2, b_l2):
    na = nodes_a.astype(jnp.int32).reshape(1, K)
    nb = nodes_b.astype(jnp.int32).reshape(1, K)
    out = _run(conn_a, conn_b, na, nb, W_e, b_e, W_n,
               b_n.reshape(1, DF), W_l1, b_l1.reshape(1, H),
               W_l2.reshape(1, H), b_l2)
    return out.reshape(K, K, 1)


# speedup vs baseline: 3.9953x; 1.0039x over previous
import functools

import jax
import jax.numpy as jnp
from jax import lax
from jax.experimental import pallas as pl
from jax.experimental.pallas import tpu as pltpu

N = 2048
K = 256
DE = 16
DF = 128
H = 128
BI = 512
T = N // BI


def _qap_body(nodes_a_ref, nodes_b_ref, conn_a_ref, conn_b_ref,
              W_e_ref, b_e_ref, W_n_ref, b_n_ref, W_l1_ref, b_l1_ref,
              W_l2_ref, b_l2_ref, out_ref,
              oh_a_ref, oh_b_ref, acc_a_ref, acc_b_ref):
    step = pl.program_id(0)

    @pl.when(step == 0)
    def _init():
        j = lax.broadcasted_iota(jnp.int32, (N, K), 0)
        oh_a_ref[...] = (j == nodes_a_ref[...]).astype(jnp.bfloat16)
        oh_b_ref[...] = (j == nodes_b_ref[...]).astype(jnp.bfloat16)
        acc_a_ref[...] = jnp.zeros_like(acc_a_ref)
        acc_b_ref[...] = jnp.zeros_like(acc_b_ref)

    def accumulate(conn_ref, oh_ref, acc_ref):
        c = conn_ref[...].astype(jnp.bfloat16)
        g = jnp.dot(c, oh_ref[...],
                    preferred_element_type=jnp.float32)
        rows = []
        for d in range(DE):
            w = W_e_ref[0, d]
            b = b_e_ref[d]
            e = jnp.maximum(g * w + b, 0.0)
            rows.append(jnp.sum(e, axis=0, keepdims=True))
        acc_ref[...] += jnp.concatenate(rows, axis=0)

    accumulate(conn_a_ref, oh_a_ref, acc_a_ref)
    accumulate(conn_b_ref, oh_b_ref, acc_b_ref)

    @pl.when(step == T - 1)
    def _head():
        dn = (((0,), (0,)), ((), ()))
        base_a = jnp.maximum(
            lax.dot_general(acc_a_ref[...], W_n_ref[...], dn,
                            preferred_element_type=jnp.float32)
            + b_n_ref[...], 0.0)
        base_b = jnp.maximum(
            lax.dot_general(acc_b_ref[...], W_n_ref[...], dn,
                            preferred_element_type=jnp.float32)
            + b_n_ref[...], 0.0)
        w_a = W_l1_ref[0:DF, :]
        w_b = W_l1_ref[DF:2 * DF, :]
        pa = jnp.dot(base_a, w_a,
                     preferred_element_type=jnp.float32) + b_l1_ref[...]
        dnt = (((0,), (1,)), ((), ()))
        pbt = lax.dot_general(w_b, base_b, dnt,
                              preferred_element_type=jnp.float32)
        acc = jnp.full((K, K), b_l2_ref[0], dtype=jnp.float32)
        for f in range(H):
            hmat = jnp.maximum(pa[:, f:f + 1] + pbt[f:f + 1, :], 0.0)
            acc = acc + W_l2_ref[0, f] * hmat
        out_ref[...] = 1.0 / (1.0 + jnp.exp(-acc))


@functools.partial(jax.jit, static_argnames=())
def _run(conn_a, conn_b, na, nb, W_e, b_e, W_n, b_n2, W_l1, b_l1_2, w2row, b_l2):
    in_specs = [
            pl.BlockSpec((1, K), lambda i: (0, 0)),
            pl.BlockSpec((1, K), lambda i: (0, 0)),
            pl.BlockSpec((BI, N), lambda i: (i, 0)),
            pl.BlockSpec((BI, N), lambda i: (i, 0)),
            pl.BlockSpec(memory_space=pltpu.SMEM),
            pl.BlockSpec(memory_space=pltpu.SMEM),
            pl.BlockSpec((DE, DF), lambda i: (0, 0)),
            pl.BlockSpec((1, DF), lambda i: (0, 0)),
            pl.BlockSpec((2 * DF, H), lambda i: (0, 0)),
            pl.BlockSpec((1, H), lambda i: (0, 0)),
            pl.BlockSpec(memory_space=pltpu.SMEM),
            pl.BlockSpec(memory_space=pltpu.SMEM),
    ]
    return pl.pallas_call(
        _qap_body,
        grid=(T,),
        in_specs=in_specs,
        out_specs=pl.BlockSpec((K, K), lambda i: (0, 0)),
        out_shape=jax.ShapeDtypeStruct((K, K), jnp.float32),
        scratch_shapes=[
            pltpu.VMEM((N, K), jnp.bfloat16),
            pltpu.VMEM((N, K), jnp.bfloat16),
            pltpu.VMEM((DE, K), jnp.float32),
            pltpu.VMEM((DE, K), jnp.float32),
        ],
        compiler_params=pltpu.CompilerParams(
            dimension_semantics=("arbitrary",),
        ),
    )(na, nb, conn_a, conn_b, W_e, b_e, W_n, b_n2, W_l1, b_l1_2, w2row, b_l2)


def kernel(conn_a, conn_b, nodes_a, nodes_b, W_e, b_e, W_n, b_n,
           W_l1, b_l1, W_l2, b_l2):
    na = nodes_a.astype(jnp.int32).reshape(1, K)
    nb = nodes_b.astype(jnp.int32).reshape(1, K)
    out = _run(conn_a, conn_b, na, nb, W_e, b_e, W_n,
               b_n.reshape(1, DF), W_l1, b_l1.reshape(1, H),
               W_l2.reshape(1, H), b_l2)
    return out.reshape(K, K, 1)
